# Initial kernel scaffold; baseline (speedup 1.0000x reference)
import jax
import jax.numpy as jnp
from jax.experimental import pallas as pl


def kernel(uv, Z):
    raise NotImplementedError("write your pallas kernel here")


# speedup vs baseline: 1.1771x; 1.1771x over previous
import functools

import jax
import jax.numpy as jnp
from jax import lax
from jax.experimental import pallas as pl
from jax.experimental.pallas import tpu as pltpu
from jax.experimental.pallas import tpu_sc as plsc

H = 2048
W = 2048
C = 16
B = 1048576

_NC = 2
_NS = 16
_NW = _NC * _NS
_BPW = B // _NW
_CH = 512
_G = _BPW // _CH
_L = 16


def _sc_body(u_hbm, v_hbm, z_hbm, out_hbm,
             ubuf, vbuf,
             ib0, ib1, ib2, ib3,
             wb0, wb1, wb2, wb3,
             tp0, tp1, tp2, tp3,
             outbuf, sem):
    wid = lax.axis_index("s") * _NC + lax.axis_index("c")
    base = wid * _BPW

    def chunk_body(g, _):
        off = base + g * _CH
        pltpu.sync_copy(u_hbm.at[pl.ds(off, _CH)], ubuf)
        pltpu.sync_copy(v_hbm.at[pl.ds(off, _CH)], vbuf)

        def grp(i, _):
            s = i * _L
            u = ubuf[pl.ds(s, _L)]
            v = vbuf[pl.ds(s, _L)]
            gx = u * 2.0 - 1.0
            gy = v * 2.0 - 1.0
            ix = ((gx + 1.0) * W - 1.0) * 0.5
            iy = ((gy + 1.0) * H - 1.0) * 0.5
            ix = jnp.minimum(jnp.maximum(ix, 0.0), W - 1.0)
            iy = jnp.minimum(jnp.maximum(iy, 0.0), H - 1.0)
            x0 = ix.astype(jnp.int32)
            y0 = iy.astype(jnp.int32)
            wx1 = ix - x0.astype(jnp.float32)
            wy1 = iy - y0.astype(jnp.float32)
            wx0 = 1.0 - wx1
            wy0 = 1.0 - wy1
            x1 = jnp.minimum(x0 + 1, W - 1)
            y1 = jnp.minimum(y0 + 1, H - 1)
            r0 = y0 * W
            r1 = y1 * W
            ib0[pl.ds(s, _L)] = r0 + x0
            ib1[pl.ds(s, _L)] = r0 + x1
            ib2[pl.ds(s, _L)] = r1 + x0
            ib3[pl.ds(s, _L)] = r1 + x1
            wb0[pl.ds(s, _L)] = wy0 * wx0
            wb1[pl.ds(s, _L)] = wy0 * wx1
            wb2[pl.ds(s, _L)] = wy1 * wx0
            wb3[pl.ds(s, _L)] = wy1 * wx1
            return 0

        lax.fori_loop(0, _CH // _L, grp, 0)

        c0 = pltpu.async_copy(z_hbm.at[ib0], tp0, sem)
        c1 = pltpu.async_copy(z_hbm.at[ib1], tp1, sem)
        c2 = pltpu.async_copy(z_hbm.at[ib2], tp2, sem)
        c3 = pltpu.async_copy(z_hbm.at[ib3], tp3, sem)
        c0.wait()
        c1.wait()
        c2.wait()
        c3.wait()

        dnums = lax.GatherDimensionNumbers(
            offset_dims=(), collapsed_slice_dims=(0,), start_index_map=(0,))

        def _splat(vec, j):
            idxs = jnp.full((_L, 1), j, dtype=jnp.int32)
            return lax.gather(vec, idxs, dnums, slice_sizes=(1,),
                              mode=lax.GatherScatterMode.PROMISE_IN_BOUNDS)

        def grp2(i, _):
            s = i * _L
            w0 = wb0[pl.ds(s, _L)]
            w1 = wb1[pl.ds(s, _L)]
            w2 = wb2[pl.ds(s, _L)]
            w3 = wb3[pl.ds(s, _L)]
            for j in range(_L):
                p = s + j
                acc = (tp0[p, :] * _splat(w0, j) + tp1[p, :] * _splat(w1, j)
                       + tp2[p, :] * _splat(w2, j) + tp3[p, :] * _splat(w3, j))
                outbuf[p, :] = acc
            return 0

        lax.fori_loop(0, _CH // _L, grp2, 0)
        pltpu.sync_copy(outbuf, out_hbm.at[pl.ds(off, _CH)])
        return 0

    lax.fori_loop(0, _G, chunk_body, 0)


_sc_call = functools.partial(
    pl.kernel,
    mesh=plsc.VectorSubcoreMesh(core_axis_name="c", subcore_axis_name="s"),
    out_type=jax.ShapeDtypeStruct((B, C), jnp.float32),
    compiler_params=pltpu.CompilerParams(use_tc_tiling_on_sc=False),
    scratch_types=(
        [pltpu.VMEM((_CH,), jnp.float32)] * 2
        + [pltpu.VMEM((_CH,), jnp.int32)] * 4
        + [pltpu.VMEM((_CH,), jnp.float32)] * 4
        + [pltpu.VMEM((_CH, C), jnp.float32)] * 4
        + [pltpu.VMEM((_CH, C), jnp.float32)]
        + [pltpu.SemaphoreType.DMA]
    ),
)(_sc_body)


def kernel(uv, Z):
    u = uv[:, 0]
    v = uv[:, 1]
    zr = jnp.transpose(Z[0], (1, 2, 0)).reshape(H * W, C)
    return _sc_call(u, v, zr)
